# baseline (speedup 1.0000x reference)
import functools

import jax
import jax.numpy as jnp
from jax import lax
from jax.experimental import pallas as pl
from jax.experimental.pallas import tpu as pltpu
from jax.experimental.pallas import tpu_sc as plsc

B = 1024
S = 200
N_POS = 53
N_NER = 25
POS_DIM = 30
NER_DIM = 30

_RBLK = 8
_SBLK = 8


def _mask_body(masks_ref, head_ref, adj_ref):
    i = pl.program_id(0)
    l = jnp.sum((masks_ref[...] == 0.0).astype(jnp.int32), axis=0, keepdims=True)
    head2 = head_ref[...]
    cvec2 = jax.lax.broadcasted_iota(jnp.int32, (S, 1), 0)
    col_valid2 = cvec2 < l
    head_eff = jnp.where((head2 > 0) & col_valid2, head2, 0)
    rvec = jax.lax.broadcasted_iota(jnp.int32, (_RBLK, 1, 1), 0) + i * _RBLK
    eq = head_eff[None, :, :] == rvec + 1
    adj_ref[...] = eq.astype(jnp.float32)


_DP = 32
_NW = 32
_TOK = B * S
_PER_W = _TOK // _NW
_CH = 3200


def _sc_emb_body(pt_hbm, nt_hbm, pidx_hbm, nidx_hbm, pout_hbm, nout_hbm,
                 idx_v, rows_v, sem):
    wid = lax.axis_index("s") * 2 + lax.axis_index("c")
    for tab_hbm, idx_hbm, out_hbm in ((pt_hbm, pidx_hbm, pout_hbm),
                                      (nt_hbm, nidx_hbm, nout_hbm)):
        for ch in range(_PER_W // _CH):
            base = wid * _PER_W + ch * _CH
            pltpu.sync_copy(idx_hbm.at[pl.ds(base, _CH)], idx_v)
            pltpu.async_copy(tab_hbm.at[idx_v], rows_v, sem).wait()
            pltpu.sync_copy(rows_v, out_hbm.at[pl.ds(base, _CH)])


def kernel(words, masks, pos, ner, deprel, head, subj_pos, obj_pos, subj_type, obj_type,
           pos_table, ner_table):
    del words, deprel, subj_pos, obj_pos, subj_type, obj_type
    masks_t = masks.T
    head_t = head.T
    pos_t = pos.T
    ner_t = ner.T

    adj_t = pl.pallas_call(
        _mask_body,
        grid=(S // _RBLK,),
        in_specs=[
            pl.BlockSpec((S, B), lambda i: (0, 0)),
            pl.BlockSpec((S, B), lambda i: (0, 0)),
        ],
        out_specs=pl.BlockSpec((_RBLK, S, B), lambda i: (i, 0, 0)),
        out_shape=jax.ShapeDtypeStruct((S, S, B), jnp.float32),
    )(masks_t, head_t)

    adj = jnp.transpose(adj_t, (2, 0, 1))

    mesh = plsc.VectorSubcoreMesh(core_axis_name="c", subcore_axis_name="s")
    sc_emb = pl.kernel(
        _sc_emb_body,
        [
            jax.ShapeDtypeStruct((_TOK, _DP), jnp.float32),
            jax.ShapeDtypeStruct((_TOK, _DP), jnp.float32),
        ],
        mesh=mesh,
        scratch_types=[
            pltpu.VMEM((_CH,), jnp.int32),
            pltpu.VMEM((_CH, _DP), jnp.float32),
            pltpu.SemaphoreType.DMA,
        ],
        compiler_params=pltpu.CompilerParams(use_tc_tiling_on_sc=False),
    )
    pt_pad = jnp.pad(pos_table, ((0, 0), (0, _DP - POS_DIM)))
    nt_pad = jnp.pad(ner_table, ((0, 0), (0, _DP - NER_DIM)))
    pos_ep, ner_ep = sc_emb(pt_pad, nt_pad, pos.reshape(-1), ner.reshape(-1))
    pos_embs = pos_ep.reshape(B, S, _DP)[:, :, :POS_DIM]
    ner_embs = ner_ep.reshape(B, S, _DP)[:, :, :NER_DIM]

    l = jnp.sum((masks == 0.0).astype(jnp.int32), axis=1)
    alen = jnp.arange(S)
    amask = alen[None, :] < l[:, None]
    pad_mask = jnp.broadcast_to((~amask)[:, None, :], (B, S, S))
    head_eff = jnp.where((head > 0) & amask, head, 0)
    dep_mask = head_eff[:, None, :] != (alen + 1)[None, :, None]
    seq_mask = jnp.broadcast_to(~(alen[None, None, :] <= alen[None, :, None]),
                                (B, S, S))

    return (pos_embs, ner_embs, dep_mask, pad_mask, seq_mask, adj)


# speedup vs baseline: 5.6530x; 5.6530x over previous
import jax
import jax.numpy as jnp
from jax.experimental import pallas as pl

B = 1024
S = 200
N_POS = 53
N_NER = 25
POS_DIM = 30
NER_DIM = 30

_RBLK = 8
_SBLK = 8


def _mask_body(masks_ref, head_ref, adj_ref):
    i = pl.program_id(0)
    l = jnp.sum((masks_ref[...] == 0.0).astype(jnp.int32), axis=0, keepdims=True)
    head2 = head_ref[...]
    cvec2 = jax.lax.broadcasted_iota(jnp.int32, (S, 1), 0)
    col_valid2 = cvec2 < l
    head_eff = jnp.where((head2 > 0) & col_valid2, head2, 0)
    rvec = jax.lax.broadcasted_iota(jnp.int32, (_RBLK, 1, 1), 0) + i * _RBLK
    eq = head_eff[None, :, :] == rvec + 1
    adj_ref[...] = eq.astype(jnp.float32)


_KCAT = N_POS + N_NER
_MCAT = 64


def _emb_body(pos_ref, ner_ref, cat_ref, pos_out, ner_out):
    cat = cat_ref[...]
    kk = jax.lax.broadcasted_iota(jnp.int32, (_KCAT, 1), 0)
    for s in range(_SBLK):
        prow = pos_ref[s:s + 1, :]
        nrow = ner_ref[s:s + 1, :]
        sel = jnp.where(kk < N_POS, prow, nrow + N_POS)
        oh = (sel == kk).astype(jnp.float32)
        res = jnp.dot(cat, oh, preferred_element_type=jnp.float32)
        pos_out[:, s:s + 1, :] = res[0:POS_DIM][:, None, :]
        ner_out[:, s:s + 1, :] = res[32:32 + NER_DIM][:, None, :]


def kernel(words, masks, pos, ner, deprel, head, subj_pos, obj_pos, subj_type, obj_type,
           pos_table, ner_table):
    del words, deprel, subj_pos, obj_pos, subj_type, obj_type
    masks_t = masks.T
    head_t = head.T
    pos_t = pos.T
    ner_t = ner.T

    adj_t = pl.pallas_call(
        _mask_body,
        grid=(S // _RBLK,),
        in_specs=[
            pl.BlockSpec((S, B), lambda i: (0, 0)),
            pl.BlockSpec((S, B), lambda i: (0, 0)),
        ],
        out_specs=pl.BlockSpec((_RBLK, S, B), lambda i: (i, 0, 0)),
        out_shape=jax.ShapeDtypeStruct((S, S, B), jnp.float32),
    )(masks_t, head_t)

    adj = jnp.transpose(adj_t, (2, 0, 1))

    emb_call = pl.pallas_call(
        _emb_body,
        grid=(S // _SBLK,),
        in_specs=[
            pl.BlockSpec((_SBLK, B), lambda i: (i, 0)),
            pl.BlockSpec((_SBLK, B), lambda i: (i, 0)),
            pl.BlockSpec((_MCAT, _KCAT), lambda i: (0, 0)),
        ],
        out_specs=[
            pl.BlockSpec((POS_DIM, _SBLK, B), lambda i: (0, i, 0)),
            pl.BlockSpec((NER_DIM, _SBLK, B), lambda i: (0, i, 0)),
        ],
        out_shape=[
            jax.ShapeDtypeStruct((POS_DIM, S, B), jnp.float32),
            jax.ShapeDtypeStruct((NER_DIM, S, B), jnp.float32),
        ],
    )
    cat = jnp.zeros((_MCAT, _KCAT), jnp.float32)
    cat = cat.at[0:POS_DIM, 0:N_POS].set(pos_table.T)
    cat = cat.at[32:32 + NER_DIM, N_POS:].set(ner_table.T)
    pos_et, ner_et = emb_call(pos_t, ner_t, cat)

    pos_embs = jnp.transpose(pos_et, (2, 1, 0))
    ner_embs = jnp.transpose(ner_et, (2, 1, 0))

    l = jnp.sum((masks == 0.0).astype(jnp.int32), axis=1)
    alen = jnp.arange(S)
    amask = alen[None, :] < l[:, None]
    pad_mask = jnp.broadcast_to((~amask)[:, None, :], (B, S, S))
    head_eff = jnp.where((head > 0) & amask, head, 0)
    dep_mask = head_eff[:, None, :] != (alen + 1)[None, :, None]
    seq_mask = jnp.broadcast_to(~(alen[None, None, :] <= alen[None, :, None]),
                                (B, S, S))

    return (pos_embs, ner_embs, dep_mask, pad_mask, seq_mask, adj)
